# baseline (speedup 1.0000x reference)
import jax
import jax.numpy as jnp
from jax import lax
from jax.experimental import pallas as pl
from jax.experimental.pallas import tpu as pltpu
from jax.experimental.pallas import tpu_sc as plsc

N = 10000
E = 320000
F = 128
HID = 128
NUM_CLASSES = 10
NUM_GRAPHS = 64

NC = 2
NS = 16
NW = NC * NS

EDGES_PER_TILE = E // NW
CHUNK = 128
NFULL = EDGES_PER_TILE // CHUNK
REM = EDGES_PER_TILE - NFULL * CHUNK
NP = N
ROWS_PER_TILE = NP // NS

NBUF = 3
NBODY = NFULL // NBUF
assert NFULL == NBODY * NBUF


def _seg_sum_body(h_hbm, src_hbm, dst_hbm, zeros_hbm, out_hbm,
                  srcs, dsts, rows, src_r, dst_r, acc_sh,
                  semsrc, semdst, semg, sem_r):
    c = lax.axis_index("c")
    s = lax.axis_index("s")
    wid = s * NC + c
    edge_base = wid * EDGES_PER_TILE

    r0 = s * ROWS_PER_TILE
    pltpu.sync_copy(zeros_hbm.at[pl.ds(r0, ROWS_PER_TILE)],
                    acc_sh.at[pl.ds(r0, ROWS_PER_TILE)])
    plsc.subcore_barrier()

    def issue_idx(i, b):
        pltpu.async_copy(src_hbm.at[pl.ds(edge_base + i * CHUNK, CHUNK)],
                         srcs[b], semsrc[b])
        pltpu.async_copy(dst_hbm.at[pl.ds(edge_base + i * CHUNK, CHUNK)],
                         dsts[b], semdst[b])

    def wait_idx_src(b):
        pltpu.make_async_copy(src_hbm.at[pl.ds(0, CHUNK)], srcs[b],
                              semsrc[b]).wait()

    def wait_idx_dst(b):
        pltpu.make_async_copy(dst_hbm.at[pl.ds(0, CHUNK)], dsts[b],
                              semdst[b]).wait()

    def issue_gather(b):
        pltpu.async_copy(h_hbm.at[srcs[b]], rows[b], semg[b])

    def wait_gather(b):
        pltpu.make_async_copy(h_hbm.at[srcs[b]], rows[b], semg[b]).wait()

    for b in range(NBUF):
        issue_idx(b, b)
    wait_idx_src(0)
    issue_gather(0)

    def body(j, carry):
        i0 = NBUF * j
        for b in range(NBUF):
            i = i0 + b
            b1 = (b + 1) % NBUF
            wait_gather(b)

            @pl.when(i + 1 < NFULL)
            def _():
                wait_idx_src(b1)
                issue_gather(b1)

            wait_idx_dst(b)
            pltpu.sync_copy(rows[b], acc_sh.at[dsts[b]], add=True)

            @pl.when(i + NBUF < NFULL)
            def _():
                issue_idx(i + NBUF, b)
        return carry

    lax.fori_loop(0, NBODY, body, 0)

    base = NFULL * CHUNK
    pltpu.sync_copy(src_hbm.at[pl.ds(edge_base + base, REM)], src_r)
    pltpu.sync_copy(dst_hbm.at[pl.ds(edge_base + base, REM)], dst_r)
    rows_r = rows[0].at[pl.ds(0, REM)]
    pltpu.async_copy(h_hbm.at[src_r], rows_r, sem_r).wait()
    pltpu.sync_copy(rows_r, acc_sh.at[dst_r], add=True)

    plsc.subcore_barrier()
    pltpu.sync_copy(acc_sh.at[pl.ds(r0, ROWS_PER_TILE)],
                    out_hbm.at[c, pl.ds(r0, ROWS_PER_TILE)])


@jax.jit
def _segment_sum_sc(h, src, dst, zeros):
    mesh = plsc.VectorSubcoreMesh(core_axis_name="c", subcore_axis_name="s",
                                  num_cores=NC, num_subcores=NS)
    return pl.kernel(
        _seg_sum_body,
        out_type=jax.ShapeDtypeStruct((NC, NP, F), jnp.float32),
        mesh=mesh,
        compiler_params=pltpu.CompilerParams(use_tc_tiling_on_sc=False),
        scratch_types=[
            tuple(pltpu.VMEM((CHUNK,), jnp.int32) for _ in range(NBUF)),
            tuple(pltpu.VMEM((CHUNK,), jnp.int32) for _ in range(NBUF)),
            tuple(pltpu.VMEM((CHUNK, F), jnp.float32) for _ in range(NBUF)),
            pltpu.VMEM((REM,), jnp.int32),
            pltpu.VMEM((REM,), jnp.int32),
            pltpu.VMEM_SHARED((NP, F), jnp.float32),
            tuple(pltpu.SemaphoreType.DMA for _ in range(NBUF)),
            tuple(pltpu.SemaphoreType.DMA for _ in range(NBUF)),
            tuple(pltpu.SemaphoreType.DMA for _ in range(NBUF)),
            pltpu.SemaphoreType.DMA,
        ],
    )(h, src, dst, zeros)


def _mlp_body(h_ref, part_ref, eps_ref, w1_ref, b1_ref, w2_ref, b2_ref,
              gamma_ref, beta_ref, out_ref):
    z = ((1.0 + eps_ref[0, 0]) * h_ref[...]
         + part_ref[0, :N, :] + part_ref[1, :N, :])
    a = jnp.dot(z, w1_ref[...], preferred_element_type=jnp.float32,
                precision=lax.Precision.HIGHEST)
    a = jnp.maximum(a + b1_ref[...], 0.0)
    a = jnp.dot(a, w2_ref[...], preferred_element_type=jnp.float32,
                precision=lax.Precision.HIGHEST)
    a = jnp.maximum(a + b2_ref[...], 0.0)
    mean = jnp.mean(a, axis=0, keepdims=True)
    var = jnp.mean((a - mean) * (a - mean), axis=0, keepdims=True)
    out_ref[...] = ((a - mean) * lax.rsqrt(var + 1e-5) * gamma_ref[...]
                    + beta_ref[...])


@jax.jit
def _mlp_tc(h, part, eps, w1, b1, w2, b2, gamma, beta):
    return pl.pallas_call(
        _mlp_body,
        out_shape=jax.ShapeDtypeStruct((N, HID), jnp.float32),
    )(h, part, eps.reshape(1, 1), w1, b1.reshape(1, HID), w2,
      b2.reshape(1, HID), gamma.reshape(1, HID), beta.reshape(1, HID))


def _head_body(h_ref, batch_ref, w1_ref, b1_ref, w2_ref, b2_ref, out_ref):
    gids = lax.broadcasted_iota(jnp.int32, (N, NUM_GRAPHS), 1)
    oh = (batch_ref[...] == gids).astype(jnp.float32)
    sums = lax.dot_general(oh, h_ref[...], (((0,), (0,)), ((), ())),
                           preferred_element_type=jnp.float32,
                           precision=lax.Precision.HIGHEST)
    counts = jnp.sum(oh, axis=0, keepdims=True)
    g = sums / jnp.maximum(counts, 1.0).T
    g = jnp.dot(g, w1_ref[...], preferred_element_type=jnp.float32,
                precision=lax.Precision.HIGHEST)
    g = jnp.maximum(g + b1_ref[...], 0.0)
    g = jnp.dot(g, w2_ref[...], preferred_element_type=jnp.float32,
                precision=lax.Precision.HIGHEST)
    logits = g + b2_ref[...]
    m = jnp.max(logits, axis=-1, keepdims=True)
    lse = m + jnp.log(jnp.sum(jnp.exp(logits - m), axis=-1, keepdims=True))
    out_ref[...] = logits - lse


@jax.jit
def _head_tc(h, batch, w1, b1, w2, b2):
    return pl.pallas_call(
        _head_body,
        out_shape=jax.ShapeDtypeStruct((NUM_GRAPHS, NUM_CLASSES), jnp.float32),
    )(h, batch.reshape(N, 1), w1, b1.reshape(1, HID), w2,
      b2.reshape(1, NUM_CLASSES))


def kernel(x, edge_index, batch, params):
    src = edge_index[0]
    dst = edge_index[1]
    zeros = jnp.zeros((NP, F), jnp.float32)
    h = x
    for l in range(3):
        p = params['conv%d' % l]
        part = _segment_sum_sc(h, src, dst, zeros)
        h = _mlp_tc(h, part, p['eps'], p['W1'], p['b1'], p['W2'], p['b2'],
                    p['gamma'], p['beta'])
    return _head_tc(h, batch, params['lin1']['W'], params['lin1']['b'],
                    params['lin2']['W'], params['lin2']['b'])


# speedup vs baseline: 1.2696x; 1.2696x over previous
import jax
import jax.numpy as jnp
from jax import lax
from jax.experimental import pallas as pl
from jax.experimental.pallas import tpu as pltpu
from jax.experimental.pallas import tpu_sc as plsc

N = 10000
E = 320000
F = 128
HID = 128
NUM_CLASSES = 10
NUM_GRAPHS = 64

NC = 2
NS = 16
NW = NC * NS

EDGES_PER_TILE = E // NW
CHUNK = 128
NFULL = EDGES_PER_TILE // CHUNK
REM = EDGES_PER_TILE - NFULL * CHUNK
NP = 10240
ROWS_PER_TILE = NP // NS

NBUF = 2
NBODY = NFULL // NBUF
assert NFULL == NBODY * NBUF


def _seg_sum_body(h_hbm, src_hbm, dst_hbm, zeros_hbm, out_hbm,
                  src_all, dsts, rows, src_r, dst_r, rows_r, acc_sh,
                  sems, semds, sem_r):
    c = lax.axis_index("c")
    s = lax.axis_index("s")
    wid = s * NC + c
    edge_base = wid * EDGES_PER_TILE

    pltpu.sync_copy(src_hbm.at[pl.ds(edge_base, EDGES_PER_TILE)], src_all)

    r0 = s * ROWS_PER_TILE
    pltpu.sync_copy(zeros_hbm.at[pl.ds(r0, ROWS_PER_TILE)],
                    acc_sh.at[pl.ds(r0, ROWS_PER_TILE)])
    plsc.subcore_barrier()

    def issue(i, b):
        pltpu.async_copy(dst_hbm.at[pl.ds(edge_base + i * CHUNK, CHUNK)],
                         dsts[b], semds[b])
        pltpu.async_copy(h_hbm.at[src_all.at[pl.ds(i * CHUNK, CHUNK)]],
                         rows[b], sems[b])

    def drain(b):
        pltpu.make_async_copy(h_hbm.at[src_all.at[pl.ds(0, CHUNK)]],
                              rows[b], sems[b]).wait()
        pltpu.make_async_copy(dst_hbm.at[pl.ds(0, CHUNK)], dsts[b],
                              semds[b]).wait()
        pltpu.sync_copy(rows[b], acc_sh.at[dsts[b]], add=True)

    for b in range(NBUF):
        issue(b, b)

    def body(j, carry):
        i0 = NBUF * j
        for b in range(NBUF):
            drain(b)

            @pl.when(i0 + b + NBUF < NFULL)
            def _():
                issue(i0 + b + NBUF, b)
        return carry

    lax.fori_loop(0, NBODY, body, 0)

    base = NFULL * CHUNK
    pltpu.sync_copy(src_hbm.at[pl.ds(edge_base + base, REM)], src_r)
    pltpu.sync_copy(dst_hbm.at[pl.ds(edge_base + base, REM)], dst_r)
    pltpu.async_copy(h_hbm.at[src_r], rows_r, sem_r).wait()
    pltpu.sync_copy(rows_r, acc_sh.at[dst_r], add=True)

    plsc.subcore_barrier()
    pltpu.sync_copy(acc_sh.at[pl.ds(r0, ROWS_PER_TILE)],
                    out_hbm.at[c, pl.ds(r0, ROWS_PER_TILE)])


@jax.jit
def _segment_sum_sc(h, src, dst, zeros):
    mesh = plsc.VectorSubcoreMesh(core_axis_name="c", subcore_axis_name="s",
                                  num_cores=NC, num_subcores=NS)
    return pl.kernel(
        _seg_sum_body,
        out_type=jax.ShapeDtypeStruct((NC, NP, F), jnp.float32),
        mesh=mesh,
        scratch_types=[
            pltpu.VMEM((EDGES_PER_TILE,), jnp.int32),
            tuple(pltpu.VMEM((CHUNK,), jnp.int32) for _ in range(NBUF)),
            tuple(pltpu.VMEM((CHUNK, F), jnp.float32) for _ in range(NBUF)),
            pltpu.VMEM((REM,), jnp.int32),
            pltpu.VMEM((REM,), jnp.int32),
            pltpu.VMEM((REM, F), jnp.float32),
            pltpu.VMEM_SHARED((NP, F), jnp.float32),
            tuple(pltpu.SemaphoreType.DMA for _ in range(NBUF)),
            tuple(pltpu.SemaphoreType.DMA for _ in range(NBUF)),
            pltpu.SemaphoreType.DMA,
        ],
    )(h, src, dst, zeros)


def _mlp_body(h_ref, part_ref, eps_ref, w1_ref, b1_ref, w2_ref, b2_ref,
              gamma_ref, beta_ref, out_ref):
    z = ((1.0 + eps_ref[0, 0]) * h_ref[...]
         + part_ref[0, :N, :] + part_ref[1, :N, :])
    a = jnp.dot(z, w1_ref[...], preferred_element_type=jnp.float32)
    a = jnp.maximum(a + b1_ref[...], 0.0)
    a = jnp.dot(a, w2_ref[...], preferred_element_type=jnp.float32)
    a = jnp.maximum(a + b2_ref[...], 0.0)
    mean = jnp.mean(a, axis=0, keepdims=True)
    var = jnp.mean((a - mean) * (a - mean), axis=0, keepdims=True)
    out_ref[...] = ((a - mean) * lax.rsqrt(var + 1e-5) * gamma_ref[...]
                    + beta_ref[...])


@jax.jit
def _mlp_tc(h, part, eps, w1, b1, w2, b2, gamma, beta):
    return pl.pallas_call(
        _mlp_body,
        out_shape=jax.ShapeDtypeStruct((N, HID), jnp.float32),
    )(h, part, eps.reshape(1, 1), w1, b1.reshape(1, HID), w2,
      b2.reshape(1, HID), gamma.reshape(1, HID), beta.reshape(1, HID))


def _head_body(h_ref, batch_ref, w1_ref, b1_ref, w2_ref, b2_ref, out_ref):
    gids = lax.broadcasted_iota(jnp.int32, (N, NUM_GRAPHS), 1)
    oh = (batch_ref[...] == gids).astype(jnp.float32)
    sums = lax.dot_general(oh, h_ref[...], (((0,), (0,)), ((), ())),
                           preferred_element_type=jnp.float32)
    counts = jnp.sum(oh, axis=0, keepdims=True)
    g = sums / jnp.maximum(counts, 1.0).T
    g = jnp.dot(g, w1_ref[...], preferred_element_type=jnp.float32)
    g = jnp.maximum(g + b1_ref[...], 0.0)
    g = jnp.dot(g, w2_ref[...], preferred_element_type=jnp.float32)
    logits = g + b2_ref[...]
    m = jnp.max(logits, axis=-1, keepdims=True)
    lse = m + jnp.log(jnp.sum(jnp.exp(logits - m), axis=-1, keepdims=True))
    out_ref[...] = logits - lse


@jax.jit
def _head_tc(h, batch, w1, b1, w2, b2):
    return pl.pallas_call(
        _head_body,
        out_shape=jax.ShapeDtypeStruct((NUM_GRAPHS, NUM_CLASSES), jnp.float32),
    )(h, batch.reshape(N, 1), w1, b1.reshape(1, HID), w2,
      b2.reshape(1, NUM_CLASSES))


def kernel(x, edge_index, batch, params):
    src = edge_index[0]
    dst = edge_index[1]
    zeros = jnp.zeros((NP, F), jnp.float32)
    h = x
    for l in range(3):
        p = params['conv%d' % l]
        part = _segment_sum_sc(h, src, dst, zeros)
        h = _mlp_tc(h, part, p['eps'], p['W1'], p['b1'], p['W2'], p['b2'],
                    p['gamma'], p['beta'])
    return _head_tc(h, batch, params['lin1']['W'], params['lin1']['b'],
                    params['lin2']['W'], params['lin2']['b'])


# speedup vs baseline: 1.4264x; 1.1235x over previous
import jax
import jax.numpy as jnp
from jax import lax
from jax.experimental import pallas as pl
from jax.experimental.pallas import tpu as pltpu
from jax.experimental.pallas import tpu_sc as plsc

N = 10000
E = 320000
F = 128
HID = 128
NUM_CLASSES = 10
NUM_GRAPHS = 64

NC = 2
NS = 16
NW = NC * NS

EDGES_PER_TILE = E // NW
CHUNK = 96
NFULL = EDGES_PER_TILE // CHUNK
REM = EDGES_PER_TILE - NFULL * CHUNK
NP = 10240
ROWS_PER_TILE = NP // NS

NBUF = 3
NBODY = NFULL // NBUF
NTAIL = NFULL - NBODY * NBUF


def _seg_sum_body(h_hbm, src_hbm, dst_hbm, zeros_hbm, out_hbm,
                  src_all, dsts, rows, src_r, dst_r, acc_sh,
                  sems, semds, sem_r):
    c = lax.axis_index("c")
    s = lax.axis_index("s")
    wid = s * NC + c
    edge_base = wid * EDGES_PER_TILE

    r0 = s * ROWS_PER_TILE
    zero_copy = pltpu.async_copy(zeros_hbm.at[pl.ds(r0, ROWS_PER_TILE)],
                                 acc_sh.at[pl.ds(r0, ROWS_PER_TILE)], sem_r)

    pltpu.sync_copy(src_hbm.at[pl.ds(edge_base, EDGES_PER_TILE)], src_all)

    def issue(i, b):
        pltpu.async_copy(dst_hbm.at[pl.ds(edge_base + i * CHUNK, CHUNK)],
                         dsts[b], semds[b])
        pltpu.async_copy(h_hbm.at[src_all.at[pl.ds(i * CHUNK, CHUNK)]],
                         rows[b], sems[b])

    def drain(b):
        pltpu.make_async_copy(h_hbm.at[src_all.at[pl.ds(0, CHUNK)]],
                              rows[b], sems[b]).wait()
        pltpu.make_async_copy(dst_hbm.at[pl.ds(0, CHUNK)], dsts[b],
                              semds[b]).wait()
        pltpu.sync_copy(rows[b], acc_sh.at[dsts[b]], add=True)

    for b in range(NBUF):
        issue(b, b)
    zero_copy.wait()
    plsc.subcore_barrier()

    def body(j, carry):
        i0 = NBUF * j
        for b in range(NBUF):
            drain(b)

            @pl.when(i0 + b + NBUF < NFULL)
            def _():
                issue(i0 + b + NBUF, b)
        return carry

    lax.fori_loop(0, NBODY, body, 0)
    for b in range(NTAIL):
        drain(b)

    base = NFULL * CHUNK
    pltpu.sync_copy(src_hbm.at[pl.ds(edge_base + base, REM)], src_r)
    pltpu.sync_copy(dst_hbm.at[pl.ds(edge_base + base, REM)], dst_r)
    rows_r = rows[0].at[pl.ds(0, REM)]
    pltpu.async_copy(h_hbm.at[src_r], rows_r, sem_r).wait()
    pltpu.sync_copy(rows_r, acc_sh.at[dst_r], add=True)

    plsc.subcore_barrier()
    pltpu.sync_copy(acc_sh.at[pl.ds(r0, ROWS_PER_TILE)],
                    out_hbm.at[c, pl.ds(r0, ROWS_PER_TILE)])


@jax.jit
def _segment_sum_sc(h, src, dst, zeros):
    mesh = plsc.VectorSubcoreMesh(core_axis_name="c", subcore_axis_name="s",
                                  num_cores=NC, num_subcores=NS)
    return pl.kernel(
        _seg_sum_body,
        out_type=jax.ShapeDtypeStruct((NC, NP, F), jnp.float32),
        mesh=mesh,
        scratch_types=[
            pltpu.VMEM((EDGES_PER_TILE,), jnp.int32),
            tuple(pltpu.VMEM((CHUNK,), jnp.int32) for _ in range(NBUF)),
            tuple(pltpu.VMEM((CHUNK, F), jnp.float32) for _ in range(NBUF)),
            pltpu.VMEM((REM,), jnp.int32),
            pltpu.VMEM((REM,), jnp.int32),
            pltpu.VMEM_SHARED((NP, F), jnp.float32),
            tuple(pltpu.SemaphoreType.DMA for _ in range(NBUF)),
            tuple(pltpu.SemaphoreType.DMA for _ in range(NBUF)),
            pltpu.SemaphoreType.DMA,
        ],
    )(h, src, dst, zeros)


def _mlp_body(h_ref, part_ref, eps_ref, w1_ref, b1_ref, w2_ref, b2_ref,
              gamma_ref, beta_ref, out_ref):
    z = ((1.0 + eps_ref[0, 0]) * h_ref[...]
         + part_ref[0, :N, :] + part_ref[1, :N, :])
    a = jnp.dot(z, w1_ref[...], preferred_element_type=jnp.float32)
    a = jnp.maximum(a + b1_ref[...], 0.0)
    a = jnp.dot(a, w2_ref[...], preferred_element_type=jnp.float32)
    a = jnp.maximum(a + b2_ref[...], 0.0)
    mean = jnp.mean(a, axis=0, keepdims=True)
    var = jnp.mean((a - mean) * (a - mean), axis=0, keepdims=True)
    out_ref[...] = ((a - mean) * lax.rsqrt(var + 1e-5) * gamma_ref[...]
                    + beta_ref[...])


@jax.jit
def _mlp_tc(h, part, eps, w1, b1, w2, b2, gamma, beta):
    return pl.pallas_call(
        _mlp_body,
        out_shape=jax.ShapeDtypeStruct((N, HID), jnp.float32),
    )(h, part, eps.reshape(1, 1), w1, b1.reshape(1, HID), w2,
      b2.reshape(1, HID), gamma.reshape(1, HID), beta.reshape(1, HID))


def _head_body(h_ref, batch_ref, w1_ref, b1_ref, w2_ref, b2_ref, out_ref):
    gids = lax.broadcasted_iota(jnp.int32, (N, NUM_GRAPHS), 1)
    oh = (batch_ref[...] == gids).astype(jnp.float32)
    sums = lax.dot_general(oh, h_ref[...], (((0,), (0,)), ((), ())),
                           preferred_element_type=jnp.float32)
    counts = jnp.sum(oh, axis=0, keepdims=True)
    g = sums / jnp.maximum(counts, 1.0).T
    g = jnp.dot(g, w1_ref[...], preferred_element_type=jnp.float32)
    g = jnp.maximum(g + b1_ref[...], 0.0)
    g = jnp.dot(g, w2_ref[...], preferred_element_type=jnp.float32)
    logits = g + b2_ref[...]
    m = jnp.max(logits, axis=-1, keepdims=True)
    lse = m + jnp.log(jnp.sum(jnp.exp(logits - m), axis=-1, keepdims=True))
    out_ref[...] = logits - lse


@jax.jit
def _head_tc(h, batch, w1, b1, w2, b2):
    return pl.pallas_call(
        _head_body,
        out_shape=jax.ShapeDtypeStruct((NUM_GRAPHS, NUM_CLASSES), jnp.float32),
    )(h, batch.reshape(N, 1), w1, b1.reshape(1, HID), w2,
      b2.reshape(1, NUM_CLASSES))


def kernel(x, edge_index, batch, params):
    src = edge_index[0]
    dst = edge_index[1]
    zeros = jnp.zeros((NP, F), jnp.float32)
    h = x
    for l in range(3):
        p = params['conv%d' % l]
        part = _segment_sum_sc(h, src, dst, zeros)
        h = _mlp_tc(h, part, p['eps'], p['W1'], p['b1'], p['W2'], p['b2'],
                    p['gamma'], p['beta'])
    return _head_tc(h, batch, params['lin1']['W'], params['lin1']['b'],
                    params['lin2']['W'], params['lin2']['b'])
